# baseline (speedup 1.0000x reference)
import functools

import jax
import jax.numpy as jnp
from jax import lax
from jax.experimental import pallas as pl
from jax.experimental.pallas import tpu as pltpu
from jax.experimental.pallas import tpu_sc as plsc

N = 10000
E = 160000
D = 256
H = 128
NS = 16
NW = 32
RPT = 624
CN = 128
CPT = 80
IDXH = CPT // 2
EP = NS * CPT * CN
CPD = EP // (NW * CN)
NBUF = 2
NTRASH = 16
DEGW = 10240
BN = 1000
GB = N // BN

_mesh = functools.partial(
    plsc.VectorSubcoreMesh, core_axis_name="c", subcore_axis_name="s")


def _deg_body(dst3d_hbm, deg_hbm, dst_all, ones_v, zdeg, deg_sh, sem):
    c = lax.axis_index("c")
    s = lax.axis_index("s")
    wid = s * 2 + c

    one16 = jnp.ones((16,), jnp.float32)
    zero16 = jnp.zeros((16,), jnp.float32)
    for t in range(CN // 16):
        ones_v[pl.ds(t * 16, 16)] = one16
    for t in range(40):
        zdeg[pl.ds(t * 16, 16)] = zero16
    pltpu.sync_copy(dst3d_hbm.at[wid], dst_all)
    pltpu.sync_copy(zdeg, deg_sh.at[pl.ds(s * 640, 640)])
    plsc.subcore_barrier()

    def fire(j, carry):
        pltpu.async_copy(ones_v, deg_sh.at[dst_all.at[j]], sem, add=True)
        return carry

    lax.fori_loop(0, CPD, fire, 0)

    def drain(j, carry):
        pltpu.make_async_copy(ones_v, deg_sh.at[dst_all.at[0]], sem).wait()
        return carry

    lax.fori_loop(0, CPD, drain, 0)
    plsc.subcore_barrier()
    pltpu.sync_copy(deg_sh.at[pl.ds(s * 640, 640)],
                    deg_hbm.at[pl.ds(c * DEGW + s * 640, 640)])


_deg_call = pl.kernel(
    _deg_body,
    out_type=jax.ShapeDtypeStruct((2 * DEGW,), jnp.float32),
    mesh=_mesh(),
    scratch_types=[
        pltpu.VMEM((CPD, CN), jnp.int32),
        pltpu.VMEM((CN,), jnp.float32),
        pltpu.VMEM((640,), jnp.float32),
        pltpu.VMEM_SHARED((DEGW,), jnp.float32),
        pltpu.SemaphoreType.DMA,
    ],
)


def _scatter_body(hsA_hbm, hsB_hbm, src_hbm, dst_hbm, accA_hbm, accB_hbm,
                  srcv0, srcv1, dstv0, dstv1, rows0, rows1,
                  zbuf, acc_sh, gsem0, gsem1, ssem0, ssem1):
    c = lax.axis_index("c")
    s = lax.axis_index("s")

    zero16 = jnp.zeros((16,), jnp.float32)

    def zrow(i, carry):
        for t in range(H // 16):
            zbuf[i, pl.ds(t * 16, 16)] = zero16
        return carry

    lax.fori_loop(0, 16, zrow, 0)
    nz = jnp.where(s == NS - 1, 40, 39)

    def zcopy(j, carry):
        pltpu.sync_copy(zbuf, acc_sh.at[pl.ds(s * RPT + j * 16, 16)])
        return carry

    lax.fori_loop(0, nz, zcopy, 0)
    plsc.subcore_barrier()

    base0 = s * CPT * CN

    def run(hs_hbm, acc_hbm):
        pltpu.sync_copy(src_hbm.at[pl.ds(base0, CN)], srcv0)
        pltpu.sync_copy(dst_hbm.at[pl.ds(base0, CN)], dstv0)
        pltpu.async_copy(hs_hbm.at[srcv0], rows0, gsem0)
        pltpu.sync_copy(src_hbm.at[pl.ds(base0 + CN, CN)], srcv1)
        pltpu.sync_copy(dst_hbm.at[pl.ds(base0 + CN, CN)], dstv1)
        pltpu.async_copy(hs_hbm.at[srcv1], rows1, gsem1)

        def pair(j, carry):
            pltpu.make_async_copy(hs_hbm.at[srcv0], rows0, gsem0).wait()
            pltpu.async_copy(rows0, acc_sh.at[dstv0], ssem0, add=True)
            pltpu.make_async_copy(hs_hbm.at[srcv1], rows1, gsem1).wait()
            pltpu.async_copy(rows1, acc_sh.at[dstv1], ssem1, add=True)

            @pl.when(j + 2 < CPT)
            def _():
                pltpu.make_async_copy(rows0, acc_sh.at[dstv0], ssem0).wait()
                base = base0 + (j + 2) * CN
                pltpu.sync_copy(src_hbm.at[pl.ds(base, CN)], srcv0)
                pltpu.sync_copy(dst_hbm.at[pl.ds(base, CN)], dstv0)
                pltpu.async_copy(hs_hbm.at[srcv0], rows0, gsem0)

            @pl.when(j + 3 < CPT)
            def _():
                pltpu.make_async_copy(rows1, acc_sh.at[dstv1], ssem1).wait()
                base = base0 + (j + 3) * CN
                pltpu.sync_copy(src_hbm.at[pl.ds(base, CN)], srcv1)
                pltpu.sync_copy(dst_hbm.at[pl.ds(base, CN)], dstv1)
                pltpu.async_copy(hs_hbm.at[srcv1], rows1, gsem1)

            return carry

        lax.fori_loop(0, CPT // 2, lambda j, cc: pair(j * 2, cc), 0)
        pltpu.make_async_copy(rows0, acc_sh.at[dstv0], ssem0).wait()
        pltpu.make_async_copy(rows1, acc_sh.at[dstv1], ssem1).wait()

        plsc.subcore_barrier()
        pltpu.sync_copy(acc_sh.at[pl.ds(s * RPT, RPT)],
                        acc_hbm.at[pl.ds(s * RPT, RPT)])

        @pl.when(s == NS - 1)
        def _():
            pltpu.sync_copy(acc_sh.at[pl.ds(NS * RPT, N - NS * RPT)],
                            acc_hbm.at[pl.ds(NS * RPT, N - NS * RPT)])

    @pl.when(c == 0)
    def _():
        run(hsA_hbm, accA_hbm)

    @pl.when(c == 1)
    def _():
        run(hsB_hbm, accB_hbm)


_scatter_call = pl.kernel(
    _scatter_body,
    out_type=(jax.ShapeDtypeStruct((N, H), jnp.float32),
              jax.ShapeDtypeStruct((N, H), jnp.float32)),
    mesh=_mesh(),
    scratch_types=[
        pltpu.VMEM((CN,), jnp.int32),
        pltpu.VMEM((CN,), jnp.int32),
        pltpu.VMEM((CN,), jnp.int32),
        pltpu.VMEM((CN,), jnp.int32),
        pltpu.VMEM((CN, H), jnp.float32),
        pltpu.VMEM((CN, H), jnp.float32),
        pltpu.VMEM((16, H), jnp.float32),
        pltpu.VMEM_SHARED((N + NTRASH, H), jnp.float32),
        pltpu.SemaphoreType.DMA,
        pltpu.SemaphoreType.DMA,
        pltpu.SemaphoreType.DMA,
        pltpu.SemaphoreType.DMA,
    ],
)


def _mm1_body(x_ref, w_ref, dv_ref, outA_ref, outB_ref):
    h = jnp.dot(x_ref[...], w_ref[...], preferred_element_type=jnp.float32)
    dv = dv_ref[...]
    outA_ref[...] = h[:, 0:H] * dv
    outB_ref[...] = h[:, H:D] * dv


def _mm1(x, W1, dinv_bc):
    return pl.pallas_call(
        _mm1_body,
        grid=(GB,),
        in_specs=[
            pl.BlockSpec((BN, D), lambda i: (i, 0)),
            pl.BlockSpec((D, D), lambda i: (0, 0)),
            pl.BlockSpec((BN, H), lambda i: (i, 0)),
        ],
        out_specs=(pl.BlockSpec((BN, H), lambda i: (i, 0)),
                   pl.BlockSpec((BN, H), lambda i: (i, 0))),
        out_shape=(jax.ShapeDtypeStruct((N, H), jnp.float32),
                   jax.ShapeDtypeStruct((N, H), jnp.float32)),
    )(x, W1, dinv_bc)


def _layer2_body(accA, accB, hsA, hsB, dv_ref, b_ref, w_ref,
                 outA_ref, outB_ref):
    dv = dv_ref[...]
    bA = b_ref[0:1, 0:H]
    bB = b_ref[0:1, H:D]
    zA = jnp.maximum(dv * (accA[...] + hsA[...]) + bA, 0.0)
    zB = jnp.maximum(dv * (accB[...] + hsB[...]) + bB, 0.0)
    w = w_ref[...]
    h2 = (jnp.dot(zA, w[0:H, :], preferred_element_type=jnp.float32)
          + jnp.dot(zB, w[H:D, :], preferred_element_type=jnp.float32))
    outA_ref[...] = h2[:, 0:H] * dv
    outB_ref[...] = h2[:, H:D] * dv


def _layer2(accA, accB, hsA, hsB, dinv_bc, b1b, W2):
    return pl.pallas_call(
        _layer2_body,
        grid=(GB,),
        in_specs=[
            pl.BlockSpec((BN, H), lambda i: (i, 0)),
            pl.BlockSpec((BN, H), lambda i: (i, 0)),
            pl.BlockSpec((BN, H), lambda i: (i, 0)),
            pl.BlockSpec((BN, H), lambda i: (i, 0)),
            pl.BlockSpec((BN, H), lambda i: (i, 0)),
            pl.BlockSpec((8, D), lambda i: (0, 0)),
            pl.BlockSpec((D, D), lambda i: (0, 0)),
        ],
        out_specs=(pl.BlockSpec((BN, H), lambda i: (i, 0)),
                   pl.BlockSpec((BN, H), lambda i: (i, 0))),
        out_shape=(jax.ShapeDtypeStruct((N, H), jnp.float32),
                   jax.ShapeDtypeStruct((N, H), jnp.float32)),
    )(accA, accB, hsA, hsB, dinv_bc, b1b, W2)


def _final_body(accA, accB, hsA, hsB, dv_ref, b_ref, out_ref):
    dv = dv_ref[...]
    bA = b_ref[0:1, 0:H]
    bB = b_ref[0:1, H:D]
    zA = jnp.maximum(dv * (accA[...] + hsA[...]) + bA, 0.0)
    zB = jnp.maximum(dv * (accB[...] + hsB[...]) + bB, 0.0)
    m = jnp.maximum(jnp.max(zA, axis=1, keepdims=True),
                    jnp.max(zB, axis=1, keepdims=True))
    se = (jnp.sum(jnp.exp(zA - m), axis=1, keepdims=True)
          + jnp.sum(jnp.exp(zB - m), axis=1, keepdims=True))
    lse = m + jnp.log(se)
    out_ref[:, 0:H] = zA - lse
    out_ref[:, H:D] = zB - lse


def _final(accA, accB, hsA, hsB, dinv_bc, b2b):
    return pl.pallas_call(
        _final_body,
        grid=(GB,),
        in_specs=[
            pl.BlockSpec((BN, H), lambda i: (i, 0)),
            pl.BlockSpec((BN, H), lambda i: (i, 0)),
            pl.BlockSpec((BN, H), lambda i: (i, 0)),
            pl.BlockSpec((BN, H), lambda i: (i, 0)),
            pl.BlockSpec((BN, H), lambda i: (i, 0)),
            pl.BlockSpec((8, D), lambda i: (0, 0)),
        ],
        out_specs=pl.BlockSpec((BN, D), lambda i: (i, 0)),
        out_shape=jax.ShapeDtypeStruct((N, D), jnp.float32),
    )(accA, accB, hsA, hsB, dinv_bc, b2b)


def kernel(x, edge_index, W1, b1, W2, b2):
    src = edge_index[0].astype(jnp.int32)
    dst = edge_index[1].astype(jnp.int32)
    src_p = jnp.concatenate([src, jnp.zeros((EP - E,), jnp.int32)])
    dst_p = jnp.concatenate([dst, jnp.full((EP - E,), N, jnp.int32)])
    dst3d = dst_p.reshape(NW, CPD, CN)

    degh = _deg_call(dst3d)
    deg = degh[:N] + degh[DEGW:DEGW + N]
    dinv = lax.rsqrt(deg + 1.0)
    dinv_bc = jnp.broadcast_to(dinv[:, None], (N, H))
    b1b = jnp.broadcast_to(b1[None, :], (8, D))
    b2b = jnp.broadcast_to(b2[None, :], (8, D))

    hsA, hsB = _mm1(x, W1, dinv_bc)
    accA, accB = _scatter_call(hsA, hsB, src_p, dst_p)
    hs2A, hs2B = _layer2(accA, accB, hsA, hsB, dinv_bc, b1b, W2)
    acc2A, acc2B = _scatter_call(hs2A, hs2B, src_p, dst_p)
    return _final(acc2A, acc2B, hs2A, hs2B, dinv_bc, b2b)


# speedup vs baseline: 1.9488x; 1.9488x over previous
import functools

import jax
import jax.numpy as jnp
from jax import lax
from jax.experimental import pallas as pl
from jax.experimental.pallas import tpu as pltpu
from jax.experimental.pallas import tpu_sc as plsc

N = 10000
E = 160000
D = 256
H = 128
NS = 16
NW = 32
RPT = 624
CN = 128
EPT = E // NS
NFULL = EPT // CN
TAIL = EPT - NFULL * CN
CPT = 80
EP = NS * CPT * CN
CPD = EP // (NW * CN)
NTRASH = 16
DEGW = 10240
BN = 1000
GB = N // BN

_mesh = functools.partial(
    plsc.VectorSubcoreMesh, core_axis_name="c", subcore_axis_name="s")


def _deg_body(dst3d_hbm, deg_hbm, dst_all, ones_v, zdeg, deg_sh, sem):
    c = lax.axis_index("c")
    s = lax.axis_index("s")
    wid = s * 2 + c

    one16 = jnp.ones((16,), jnp.float32)
    zero16 = jnp.zeros((16,), jnp.float32)
    for t in range(CN // 16):
        ones_v[pl.ds(t * 16, 16)] = one16
    for t in range(40):
        zdeg[pl.ds(t * 16, 16)] = zero16
    pltpu.sync_copy(dst3d_hbm.at[wid], dst_all)
    pltpu.sync_copy(zdeg, deg_sh.at[pl.ds(s * 640, 640)])
    plsc.subcore_barrier()

    def fire(j, carry):
        pltpu.async_copy(ones_v, deg_sh.at[dst_all.at[j]], sem, add=True)
        return carry

    lax.fori_loop(0, CPD, fire, 0)

    def drain(j, carry):
        pltpu.make_async_copy(ones_v, deg_sh.at[dst_all.at[0]], sem).wait()
        return carry

    lax.fori_loop(0, CPD, drain, 0)
    plsc.subcore_barrier()
    pltpu.sync_copy(deg_sh.at[pl.ds(s * 640, 640)],
                    deg_hbm.at[pl.ds(c * DEGW + s * 640, 640)])


_deg_call = pl.kernel(
    _deg_body,
    out_type=jax.ShapeDtypeStruct((2 * DEGW,), jnp.float32),
    mesh=_mesh(),
    scratch_types=[
        pltpu.VMEM((CPD, CN), jnp.int32),
        pltpu.VMEM((CN,), jnp.float32),
        pltpu.VMEM((640,), jnp.float32),
        pltpu.VMEM_SHARED((DEGW,), jnp.float32),
        pltpu.SemaphoreType.DMA,
    ],
)


def _scatter_body(hsA_hbm, hsB_hbm, src_hbm, dst_hbm, accA_hbm, accB_hbm,
                  srcv0, srcv1, dstv0, dstv1, rows0, rows1,
                  srct, dstt, rowst,
                  zbuf, acc_sh, gsem0, gsem1, ssem0, ssem1):
    c = lax.axis_index("c")
    s = lax.axis_index("s")

    zero16 = jnp.zeros((16,), jnp.float32)

    def zrow(i, carry):
        for t in range(H // 16):
            zbuf[i, pl.ds(t * 16, 16)] = zero16
        return carry

    lax.fori_loop(0, 16, zrow, 0)
    nz = jnp.where(s == NS - 1, 40, 39)

    def zcopy(j, carry):
        pltpu.sync_copy(zbuf, acc_sh.at[pl.ds(s * RPT + j * 16, 16)])
        return carry

    lax.fori_loop(0, nz, zcopy, 0)
    plsc.subcore_barrier()

    base0 = s * EPT

    def run(hs_hbm, acc_hbm):
        pltpu.sync_copy(src_hbm.at[pl.ds(base0, CN)], srcv0)
        pltpu.sync_copy(dst_hbm.at[pl.ds(base0, CN)], dstv0)
        pltpu.async_copy(hs_hbm.at[srcv0], rows0, gsem0)
        pltpu.sync_copy(src_hbm.at[pl.ds(base0 + CN, CN)], srcv1)
        pltpu.sync_copy(dst_hbm.at[pl.ds(base0 + CN, CN)], dstv1)
        pltpu.async_copy(hs_hbm.at[srcv1], rows1, gsem1)

        def pair(j, carry):
            pltpu.make_async_copy(hs_hbm.at[srcv0], rows0, gsem0).wait()
            pltpu.async_copy(rows0, acc_sh.at[dstv0], ssem0, add=True)
            pltpu.make_async_copy(hs_hbm.at[srcv1], rows1, gsem1).wait()
            pltpu.async_copy(rows1, acc_sh.at[dstv1], ssem1, add=True)

            @pl.when(j + 2 < NFULL)
            def _():
                pltpu.make_async_copy(rows0, acc_sh.at[dstv0], ssem0).wait()
                base = base0 + (j + 2) * CN
                pltpu.sync_copy(src_hbm.at[pl.ds(base, CN)], srcv0)
                pltpu.sync_copy(dst_hbm.at[pl.ds(base, CN)], dstv0)
                pltpu.async_copy(hs_hbm.at[srcv0], rows0, gsem0)

            @pl.when(j + 3 < NFULL)
            def _():
                pltpu.make_async_copy(rows1, acc_sh.at[dstv1], ssem1).wait()
                base = base0 + (j + 3) * CN
                pltpu.sync_copy(src_hbm.at[pl.ds(base, CN)], srcv1)
                pltpu.sync_copy(dst_hbm.at[pl.ds(base, CN)], dstv1)
                pltpu.async_copy(hs_hbm.at[srcv1], rows1, gsem1)

            return carry

        lax.fori_loop(0, NFULL // 2, lambda j, cc: pair(j * 2, cc), 0)
        pltpu.make_async_copy(rows0, acc_sh.at[dstv0], ssem0).wait()
        pltpu.make_async_copy(rows1, acc_sh.at[dstv1], ssem1).wait()

        baset = base0 + NFULL * CN
        pltpu.sync_copy(src_hbm.at[pl.ds(baset, TAIL)], srct)
        pltpu.sync_copy(dst_hbm.at[pl.ds(baset, TAIL)], dstt)
        pltpu.async_copy(hs_hbm.at[srct], rowst, gsem0).wait()
        pltpu.sync_copy(rowst, acc_sh.at[dstt], add=True)

        plsc.subcore_barrier()
        pltpu.sync_copy(acc_sh.at[pl.ds(s * RPT, RPT)],
                        acc_hbm.at[pl.ds(s * RPT, RPT)])

        @pl.when(s == NS - 1)
        def _():
            pltpu.sync_copy(acc_sh.at[pl.ds(NS * RPT, N - NS * RPT)],
                            acc_hbm.at[pl.ds(NS * RPT, N - NS * RPT)])

    @pl.when(c == 0)
    def _():
        run(hsA_hbm, accA_hbm)

    @pl.when(c == 1)
    def _():
        run(hsB_hbm, accB_hbm)


_scatter_call = pl.kernel(
    _scatter_body,
    out_type=(jax.ShapeDtypeStruct((N, H), jnp.float32),
              jax.ShapeDtypeStruct((N, H), jnp.float32)),
    mesh=_mesh(),
    scratch_types=[
        pltpu.VMEM((CN,), jnp.int32),
        pltpu.VMEM((CN,), jnp.int32),
        pltpu.VMEM((CN,), jnp.int32),
        pltpu.VMEM((CN,), jnp.int32),
        pltpu.VMEM((CN, H), jnp.float32),
        pltpu.VMEM((CN, H), jnp.float32),
        pltpu.VMEM((TAIL,), jnp.int32),
        pltpu.VMEM((TAIL,), jnp.int32),
        pltpu.VMEM((TAIL, H), jnp.float32),
        pltpu.VMEM((16, H), jnp.float32),
        pltpu.VMEM_SHARED((N + NTRASH, H), jnp.float32),
        pltpu.SemaphoreType.DMA,
        pltpu.SemaphoreType.DMA,
        pltpu.SemaphoreType.DMA,
        pltpu.SemaphoreType.DMA,
    ],
)


def _mm1_body(x_ref, w_ref, dv_ref, outA_ref, outB_ref):
    h = jnp.dot(x_ref[...], w_ref[...], preferred_element_type=jnp.float32)
    dv = dv_ref[...]
    outA_ref[...] = h[:, 0:H] * dv
    outB_ref[...] = h[:, H:D] * dv


def _mm1(x, W1, dinv_bc):
    return pl.pallas_call(
        _mm1_body,
        grid=(GB,),
        in_specs=[
            pl.BlockSpec((BN, D), lambda i: (i, 0)),
            pl.BlockSpec((D, D), lambda i: (0, 0)),
            pl.BlockSpec((BN, H), lambda i: (i, 0)),
        ],
        out_specs=(pl.BlockSpec((BN, H), lambda i: (i, 0)),
                   pl.BlockSpec((BN, H), lambda i: (i, 0))),
        out_shape=(jax.ShapeDtypeStruct((N, H), jnp.float32),
                   jax.ShapeDtypeStruct((N, H), jnp.float32)),
    )(x, W1, dinv_bc)


def _layer2_body(accA, accB, hsA, hsB, dv_ref, b_ref, w_ref,
                 outA_ref, outB_ref):
    dv = dv_ref[...]
    bA = b_ref[0:1, 0:H]
    bB = b_ref[0:1, H:D]
    zA = jnp.maximum(dv * (accA[...] + hsA[...]) + bA, 0.0)
    zB = jnp.maximum(dv * (accB[...] + hsB[...]) + bB, 0.0)
    w = w_ref[...]
    h2 = (jnp.dot(zA, w[0:H, :], preferred_element_type=jnp.float32)
          + jnp.dot(zB, w[H:D, :], preferred_element_type=jnp.float32))
    outA_ref[...] = h2[:, 0:H] * dv
    outB_ref[...] = h2[:, H:D] * dv


def _layer2(accA, accB, hsA, hsB, dinv_bc, b1b, W2):
    return pl.pallas_call(
        _layer2_body,
        grid=(GB,),
        in_specs=[
            pl.BlockSpec((BN, H), lambda i: (i, 0)),
            pl.BlockSpec((BN, H), lambda i: (i, 0)),
            pl.BlockSpec((BN, H), lambda i: (i, 0)),
            pl.BlockSpec((BN, H), lambda i: (i, 0)),
            pl.BlockSpec((BN, H), lambda i: (i, 0)),
            pl.BlockSpec((8, D), lambda i: (0, 0)),
            pl.BlockSpec((D, D), lambda i: (0, 0)),
        ],
        out_specs=(pl.BlockSpec((BN, H), lambda i: (i, 0)),
                   pl.BlockSpec((BN, H), lambda i: (i, 0))),
        out_shape=(jax.ShapeDtypeStruct((N, H), jnp.float32),
                   jax.ShapeDtypeStruct((N, H), jnp.float32)),
    )(accA, accB, hsA, hsB, dinv_bc, b1b, W2)


def _final_body(accA, accB, hsA, hsB, dv_ref, b_ref, out_ref):
    dv = dv_ref[...]
    bA = b_ref[0:1, 0:H]
    bB = b_ref[0:1, H:D]
    zA = jnp.maximum(dv * (accA[...] + hsA[...]) + bA, 0.0)
    zB = jnp.maximum(dv * (accB[...] + hsB[...]) + bB, 0.0)
    m = jnp.maximum(jnp.max(zA, axis=1, keepdims=True),
                    jnp.max(zB, axis=1, keepdims=True))
    se = (jnp.sum(jnp.exp(zA - m), axis=1, keepdims=True)
          + jnp.sum(jnp.exp(zB - m), axis=1, keepdims=True))
    lse = m + jnp.log(se)
    out_ref[:, 0:H] = zA - lse
    out_ref[:, H:D] = zB - lse


def _final(accA, accB, hsA, hsB, dinv_bc, b2b):
    return pl.pallas_call(
        _final_body,
        grid=(GB,),
        in_specs=[
            pl.BlockSpec((BN, H), lambda i: (i, 0)),
            pl.BlockSpec((BN, H), lambda i: (i, 0)),
            pl.BlockSpec((BN, H), lambda i: (i, 0)),
            pl.BlockSpec((BN, H), lambda i: (i, 0)),
            pl.BlockSpec((BN, H), lambda i: (i, 0)),
            pl.BlockSpec((8, D), lambda i: (0, 0)),
        ],
        out_specs=pl.BlockSpec((BN, D), lambda i: (i, 0)),
        out_shape=jax.ShapeDtypeStruct((N, D), jnp.float32),
    )(accA, accB, hsA, hsB, dinv_bc, b2b)


def kernel(x, edge_index, W1, b1, W2, b2):
    src = edge_index[0].astype(jnp.int32)
    dst = edge_index[1].astype(jnp.int32)
    dst_p = jnp.concatenate([dst, jnp.full((EP - E,), N, jnp.int32)])
    dst3d = dst_p.reshape(NW, CPD, CN)

    degh = _deg_call(dst3d)
    deg = degh[:N] + degh[DEGW:DEGW + N]
    dinv = lax.rsqrt(deg + 1.0)
    dinv_bc = jnp.broadcast_to(dinv[:, None], (N, H))
    b1b = jnp.broadcast_to(b1[None, :], (8, D))
    b2b = jnp.broadcast_to(b2[None, :], (8, D))

    hsA, hsB = _mm1(x, W1, dinv_bc)
    accA, accB = _scatter_call(hsA, hsB, src, dst)
    hs2A, hs2B = _layer2(accA, accB, hsA, hsB, dinv_bc, b1b, W2)
    acc2A, acc2B = _scatter_call(hs2A, hs2B, src, dst)
    return _final(acc2A, acc2B, hs2A, hs2B, dinv_bc, b2b)


# speedup vs baseline: 2.1113x; 1.0834x over previous
import functools

import jax
import jax.numpy as jnp
from jax import lax
from jax.experimental import pallas as pl
from jax.experimental.pallas import tpu as pltpu
from jax.experimental.pallas import tpu_sc as plsc

N = 10000
E = 160000
D = 256
H = 128
NS = 16
NW = 32
RPT = 624
CN = 128
EPT = E // NS
NFULL = EPT // CN
TAIL = EPT - NFULL * CN
CPT = 80
EP = NS * CPT * CN
CPD = EP // (NW * CN)
NTRASH = 16
DEGW = 10240
BN = 1000
GB = N // BN

_mesh = functools.partial(
    plsc.VectorSubcoreMesh, core_axis_name="c", subcore_axis_name="s")


def _deg_body(dst3d_hbm, deg_hbm, dst_all, ones_v, zdeg, deg_sh, sem):
    c = lax.axis_index("c")
    s = lax.axis_index("s")
    wid = s * 2 + c

    one16 = jnp.ones((16,), jnp.float32)
    zero16 = jnp.zeros((16,), jnp.float32)
    for t in range(CN // 16):
        ones_v[pl.ds(t * 16, 16)] = one16
    for t in range(40):
        zdeg[pl.ds(t * 16, 16)] = zero16
    pltpu.sync_copy(dst3d_hbm.at[wid], dst_all)
    pltpu.sync_copy(zdeg, deg_sh.at[pl.ds(s * 640, 640)])
    plsc.subcore_barrier()

    def fire(j, carry):
        pltpu.async_copy(ones_v, deg_sh.at[dst_all.at[j]], sem, add=True)
        return carry

    lax.fori_loop(0, CPD, fire, 0)

    def drain(j, carry):
        pltpu.make_async_copy(ones_v, deg_sh.at[dst_all.at[0]], sem).wait()
        return carry

    lax.fori_loop(0, CPD, drain, 0)
    plsc.subcore_barrier()
    pltpu.sync_copy(deg_sh.at[pl.ds(s * 640, 640)],
                    deg_hbm.at[pl.ds(c * DEGW + s * 640, 640)])


_deg_call = pl.kernel(
    _deg_body,
    out_type=jax.ShapeDtypeStruct((2 * DEGW,), jnp.float32),
    mesh=_mesh(),
    scratch_types=[
        pltpu.VMEM((CPD, CN), jnp.int32),
        pltpu.VMEM((CN,), jnp.float32),
        pltpu.VMEM((640,), jnp.float32),
        pltpu.VMEM_SHARED((DEGW,), jnp.float32),
        pltpu.SemaphoreType.DMA,
    ],
)


def _scatter_body(hsA_hbm, hsB_hbm, src_hbm, dst_hbm, accA_hbm, accB_hbm,
                  srcv0, srcv1, dstv0, dstv1, rows0, rows1,
                  srct, dstt, rowst,
                  zbuf, acc_sh, gsem0, gsem1, ssem0, ssem1, isem0, isem1,
                  zsem):
    c = lax.axis_index("c")
    s = lax.axis_index("s")
    base0 = s * EPT

    def run(hs_hbm, acc_hbm):
        pltpu.sync_copy(src_hbm.at[pl.ds(base0, CN)], srcv0)
        pltpu.sync_copy(dst_hbm.at[pl.ds(base0, CN)], dstv0)
        pltpu.async_copy(hs_hbm.at[srcv0], rows0, gsem0)
        pltpu.sync_copy(src_hbm.at[pl.ds(base0 + CN, CN)], srcv1)
        pltpu.sync_copy(dst_hbm.at[pl.ds(base0 + CN, CN)], dstv1)
        pltpu.async_copy(hs_hbm.at[srcv1], rows1, gsem1)

        zero16 = jnp.zeros((16,), jnp.float32)

        def zrow(i, carry):
            for t in range(H // 16):
                zbuf[i, pl.ds(t * 16, 16)] = zero16
            return carry

        lax.fori_loop(0, 16, zrow, 0)
        nz = jnp.where(s == NS - 1, 40, 39)

        def zfire(j, carry):
            pltpu.async_copy(zbuf, acc_sh.at[pl.ds(s * RPT + j * 16, 16)],
                             zsem)
            return carry

        lax.fori_loop(0, nz, zfire, 0)

        def zdrain(j, carry):
            pltpu.make_async_copy(zbuf, acc_sh.at[pl.ds(0, 16)], zsem).wait()
            return carry

        lax.fori_loop(0, nz, zdrain, 0)
        plsc.subcore_barrier()

        def pair(j, carry):
            pltpu.make_async_copy(hs_hbm.at[srcv0], rows0, gsem0).wait()

            @pl.when(j + 2 < NFULL)
            def _():
                pltpu.async_copy(src_hbm.at[pl.ds(base0 + (j + 2) * CN, CN)],
                                 srcv0, isem0)

            pltpu.async_copy(rows0, acc_sh.at[dstv0], ssem0, add=True)
            pltpu.make_async_copy(hs_hbm.at[srcv1], rows1, gsem1).wait()

            @pl.when(j + 3 < NFULL)
            def _():
                pltpu.async_copy(src_hbm.at[pl.ds(base0 + (j + 3) * CN, CN)],
                                 srcv1, isem1)

            pltpu.async_copy(rows1, acc_sh.at[dstv1], ssem1, add=True)

            @pl.when(j + 2 < NFULL)
            def _():
                pltpu.make_async_copy(rows0, acc_sh.at[dstv0], ssem0).wait()
                pltpu.sync_copy(dst_hbm.at[pl.ds(base0 + (j + 2) * CN, CN)],
                                dstv0)
                pltpu.make_async_copy(src_hbm.at[pl.ds(0, CN)],
                                      srcv0, isem0).wait()
                pltpu.async_copy(hs_hbm.at[srcv0], rows0, gsem0)

            @pl.when(j + 3 < NFULL)
            def _():
                pltpu.make_async_copy(rows1, acc_sh.at[dstv1], ssem1).wait()
                pltpu.sync_copy(dst_hbm.at[pl.ds(base0 + (j + 3) * CN, CN)],
                                dstv1)
                pltpu.make_async_copy(src_hbm.at[pl.ds(0, CN)],
                                      srcv1, isem1).wait()
                pltpu.async_copy(hs_hbm.at[srcv1], rows1, gsem1)

            return carry

        lax.fori_loop(0, NFULL // 2, lambda j, cc: pair(j * 2, cc), 0)
        pltpu.make_async_copy(rows0, acc_sh.at[dstv0], ssem0).wait()
        pltpu.make_async_copy(rows1, acc_sh.at[dstv1], ssem1).wait()

        baset = base0 + NFULL * CN
        pltpu.sync_copy(src_hbm.at[pl.ds(baset, TAIL)], srct)
        pltpu.sync_copy(dst_hbm.at[pl.ds(baset, TAIL)], dstt)
        pltpu.async_copy(hs_hbm.at[srct], rowst, gsem0).wait()
        pltpu.sync_copy(rowst, acc_sh.at[dstt], add=True)

        plsc.subcore_barrier()
        pltpu.sync_copy(acc_sh.at[pl.ds(s * RPT, RPT)],
                        acc_hbm.at[pl.ds(s * RPT, RPT)])

        @pl.when(s == NS - 1)
        def _():
            pltpu.sync_copy(acc_sh.at[pl.ds(NS * RPT, N - NS * RPT)],
                            acc_hbm.at[pl.ds(NS * RPT, N - NS * RPT)])

    @pl.when(c == 0)
    def _():
        run(hsA_hbm, accA_hbm)

    @pl.when(c == 1)
    def _():
        run(hsB_hbm, accB_hbm)


_scatter_call = pl.kernel(
    _scatter_body,
    out_type=(jax.ShapeDtypeStruct((N, H), jnp.float32),
              jax.ShapeDtypeStruct((N, H), jnp.float32)),
    mesh=_mesh(),
    scratch_types=[
        pltpu.VMEM((CN,), jnp.int32),
        pltpu.VMEM((CN,), jnp.int32),
        pltpu.VMEM((CN,), jnp.int32),
        pltpu.VMEM((CN,), jnp.int32),
        pltpu.VMEM((CN, H), jnp.float32),
        pltpu.VMEM((CN, H), jnp.float32),
        pltpu.VMEM((TAIL,), jnp.int32),
        pltpu.VMEM((TAIL,), jnp.int32),
        pltpu.VMEM((TAIL, H), jnp.float32),
        pltpu.VMEM((16, H), jnp.float32),
        pltpu.VMEM_SHARED((N + NTRASH, H), jnp.float32),
        pltpu.SemaphoreType.DMA,
        pltpu.SemaphoreType.DMA,
        pltpu.SemaphoreType.DMA,
        pltpu.SemaphoreType.DMA,
        pltpu.SemaphoreType.DMA,
        pltpu.SemaphoreType.DMA,
        pltpu.SemaphoreType.DMA,
    ],
)


def _mm1_body(x_ref, w_ref, dv_ref, outA_ref, outB_ref):
    h = jnp.dot(x_ref[...], w_ref[...], preferred_element_type=jnp.float32)
    dv = dv_ref[...]
    outA_ref[...] = h[:, 0:H] * dv
    outB_ref[...] = h[:, H:D] * dv


def _mm1(x, W1, dinv_bc):
    return pl.pallas_call(
        _mm1_body,
        grid=(GB,),
        in_specs=[
            pl.BlockSpec((BN, D), lambda i: (i, 0)),
            pl.BlockSpec((D, D), lambda i: (0, 0)),
            pl.BlockSpec((BN, H), lambda i: (i, 0)),
        ],
        out_specs=(pl.BlockSpec((BN, H), lambda i: (i, 0)),
                   pl.BlockSpec((BN, H), lambda i: (i, 0))),
        out_shape=(jax.ShapeDtypeStruct((N, H), jnp.float32),
                   jax.ShapeDtypeStruct((N, H), jnp.float32)),
    )(x, W1, dinv_bc)


def _layer2_body(accA, accB, hsA, hsB, dv_ref, b_ref, w_ref,
                 outA_ref, outB_ref):
    dv = dv_ref[...]
    bA = b_ref[0:1, 0:H]
    bB = b_ref[0:1, H:D]
    zA = jnp.maximum(dv * (accA[...] + hsA[...]) + bA, 0.0)
    zB = jnp.maximum(dv * (accB[...] + hsB[...]) + bB, 0.0)
    w = w_ref[...]
    h2 = (jnp.dot(zA, w[0:H, :], preferred_element_type=jnp.float32)
          + jnp.dot(zB, w[H:D, :], preferred_element_type=jnp.float32))
    outA_ref[...] = h2[:, 0:H] * dv
    outB_ref[...] = h2[:, H:D] * dv


def _layer2(accA, accB, hsA, hsB, dinv_bc, b1b, W2):
    return pl.pallas_call(
        _layer2_body,
        grid=(GB,),
        in_specs=[
            pl.BlockSpec((BN, H), lambda i: (i, 0)),
            pl.BlockSpec((BN, H), lambda i: (i, 0)),
            pl.BlockSpec((BN, H), lambda i: (i, 0)),
            pl.BlockSpec((BN, H), lambda i: (i, 0)),
            pl.BlockSpec((BN, H), lambda i: (i, 0)),
            pl.BlockSpec((8, D), lambda i: (0, 0)),
            pl.BlockSpec((D, D), lambda i: (0, 0)),
        ],
        out_specs=(pl.BlockSpec((BN, H), lambda i: (i, 0)),
                   pl.BlockSpec((BN, H), lambda i: (i, 0))),
        out_shape=(jax.ShapeDtypeStruct((N, H), jnp.float32),
                   jax.ShapeDtypeStruct((N, H), jnp.float32)),
    )(accA, accB, hsA, hsB, dinv_bc, b1b, W2)


def _final_body(accA, accB, hsA, hsB, dv_ref, b_ref, out_ref):
    dv = dv_ref[...]
    bA = b_ref[0:1, 0:H]
    bB = b_ref[0:1, H:D]
    zA = jnp.maximum(dv * (accA[...] + hsA[...]) + bA, 0.0)
    zB = jnp.maximum(dv * (accB[...] + hsB[...]) + bB, 0.0)
    m = jnp.maximum(jnp.max(zA, axis=1, keepdims=True),
                    jnp.max(zB, axis=1, keepdims=True))
    se = (jnp.sum(jnp.exp(zA - m), axis=1, keepdims=True)
          + jnp.sum(jnp.exp(zB - m), axis=1, keepdims=True))
    lse = m + jnp.log(se)
    out_ref[:, 0:H] = zA - lse
    out_ref[:, H:D] = zB - lse


def _final(accA, accB, hsA, hsB, dinv_bc, b2b):
    return pl.pallas_call(
        _final_body,
        grid=(GB,),
        in_specs=[
            pl.BlockSpec((BN, H), lambda i: (i, 0)),
            pl.BlockSpec((BN, H), lambda i: (i, 0)),
            pl.BlockSpec((BN, H), lambda i: (i, 0)),
            pl.BlockSpec((BN, H), lambda i: (i, 0)),
            pl.BlockSpec((BN, H), lambda i: (i, 0)),
            pl.BlockSpec((8, D), lambda i: (0, 0)),
        ],
        out_specs=pl.BlockSpec((BN, D), lambda i: (i, 0)),
        out_shape=jax.ShapeDtypeStruct((N, D), jnp.float32),
    )(accA, accB, hsA, hsB, dinv_bc, b2b)


def kernel(x, edge_index, W1, b1, W2, b2):
    src = edge_index[0].astype(jnp.int32)
    dst = edge_index[1].astype(jnp.int32)
    dst_p = jnp.concatenate([dst, jnp.full((EP - E,), N, jnp.int32)])
    dst3d = dst_p.reshape(NW, CPD, CN)

    degh = _deg_call(dst3d)
    deg = degh[:N] + degh[DEGW:DEGW + N]
    dinv = lax.rsqrt(deg + 1.0)
    dinv_bc = jnp.broadcast_to(dinv[:, None], (N, H))
    b1b = jnp.broadcast_to(b1[None, :], (8, D))
    b2b = jnp.broadcast_to(b2[None, :], (8, D))

    hsA, hsB = _mm1(x, W1, dinv_bc)
    accA, accB = _scatter_call(hsA, hsB, src, dst)
    hs2A, hs2B = _layer2(accA, accB, hsA, hsB, dinv_bc, b1b, W2)
    acc2A, acc2B = _scatter_call(hs2A, hs2B, src, dst)
    return _final(acc2A, acc2B, hs2A, hs2B, dinv_bc, b2b)
